# baseline (speedup 1.0000x reference)
import functools

import jax
import jax.numpy as jnp
from jax import lax
from jax.experimental import pallas as pl
from jax.experimental.pallas import tpu as pltpu
from jax.experimental.pallas import tpu_sc as plsc

C = 128
K = 15
H = 32
G = 8
CPG = 16
SIGMA = 0.6
NC = 2
NS = 16
NW = NC * NS
CHUNK = 4
NSLOT = 2


def _mlp_body(f_ref, w1_ref, b1_ref, w2_ref, o_ref):
    x = jnp.dot(f_ref[...], w1_ref[...], preferred_element_type=jnp.float32)
    x = x + b1_ref[...]
    x = jnp.where(x > 0, x, 0.1 * x)
    y = jnp.dot(x, w2_ref[...], preferred_element_type=jnp.float32)
    o_ref[...] = 1.0 / (1.0 + jnp.exp(-y))


def _run_mlp(s_feats, W1, b1, W2):
    m = s_feats.shape[0]
    bm = next(b for b in (1024, 512, 256, 128, 8) if m % b == 0)
    grid = (m // bm,)
    return pl.pallas_call(
        _mlp_body,
        grid=grid,
        in_specs=[
            pl.BlockSpec((bm, C), lambda i: (i, 0)),
            pl.BlockSpec((C, C), lambda i: (0, 0)),
            pl.BlockSpec((1, C), lambda i: (0, 0)),
            pl.BlockSpec((C, K * CPG), lambda i: (0, 0)),
        ],
        out_specs=pl.BlockSpec((bm, K * CPG), lambda i: (i, 0)),
        out_shape=jax.ShapeDtypeStruct((m, K * CPG), jnp.float32),
    )(s_feats, W1, b1.reshape(1, C), W2)


def _sqrt_approx(x):
    ix = plsc.bitcast(x, jnp.int32)
    y = plsc.bitcast(lax.shift_right_logical(ix, 1) + 0x1FBD1DF6, jnp.float32)
    y = 0.5 * (y + x / y)
    y = 0.5 * (y + x / y)
    y = 0.5 * (y + x / y)
    return y


def _make_sc_kernel(n_pts, m_pad):
    rows_per_tile = m_pad // NW
    n_chunks = rows_per_tile // CHUNK
    n_outer = n_chunks // NSLOT
    modw = CHUNK * K * CPG
    mesh = plsc.VectorSubcoreMesh(core_axis_name="c", subcore_axis_name="s")

    @functools.partial(
        pl.kernel,
        mesh=mesh,
        out_type=jax.ShapeDtypeStruct((m_pad * C,), jnp.float32),
        compiler_params=pltpu.CompilerParams(needs_layout_passes=False),
        scratch_types=[
            pltpu.VMEM((n_pts,), jnp.float32),
            pltpu.VMEM((n_pts,), jnp.float32),
            pltpu.VMEM((n_pts,), jnp.float32),
            pltpu.VMEM((48,), jnp.float32),
            pltpu.VMEM((K * C,), jnp.float32),
            pltpu.VMEM((rows_per_tile * 4,), jnp.float32),
            pltpu.VMEM((rows_per_tile * H,), jnp.int32),
            pltpu.VMEM((CHUNK * H, C), jnp.float32),
            pltpu.VMEM((modw,), jnp.float32),
            pltpu.VMEM((2 * CHUNK * C,), jnp.float32),
            pltpu.VMEM((CHUNK * H,), jnp.int32),
            pltpu.VMEM((CHUNK * H,), jnp.float32),
            pltpu.SemaphoreType.DMA,
            pltpu.SemaphoreType.DMA,
            pltpu.SemaphoreType.DMA,
        ],
    )
    def sc_kernel(feats_hbm, inds_hbm, q_hbm, mod_hbm, xs_hbm, ys_hbm,
                  zs_hbm, kp_hbm, wp_hbm, out_hbm,
                  xs_v, ys_v, zs_v, kp_v, wp_v, q_all, idx_all,
                  rows_v, mod_r, out_v, bk_scr, w_scr, gsem, msem, osem):
        wid = lax.axis_index("s") * NC + lax.axis_index("c")
        base = wid * rows_per_tile

        pltpu.sync_copy(xs_hbm, xs_v)
        pltpu.sync_copy(ys_hbm, ys_v)
        pltpu.sync_copy(zs_hbm, zs_v)
        pltpu.sync_copy(kp_hbm, kp_v)
        pltpu.sync_copy(wp_hbm, wp_v)
        pltpu.sync_copy(q_hbm.at[pl.ds(base * 4, rows_per_tile * 4)], q_all)
        pltpu.sync_copy(inds_hbm.at[pl.ds(base * H, rows_per_tile * H)],
                        idx_all)

        kpxv = kp_v[pl.ds(0, 16)]
        kpyv = kp_v[pl.ds(16, 16)]
        kpzv = kp_v[pl.ds(32, 16)]
        kpx = [kpxv[k] for k in range(K)]
        kpy = [kpyv[k] for k in range(K)]
        kpz = [kpzv[k] for k in range(K)]

        lane = lax.iota(jnp.int32, 16)

        def geom_row(ci, r):
            ib = ci * CHUNK * H + r * H
            iv0 = idx_all[pl.ds(ib, 16)]
            iv1 = idx_all[pl.ds(ib + 16, 16)]
            qb = ci * CHUNK * 4 + r * 4
            qx = plsc.load_gather(q_all, [jnp.full((16,), qb, jnp.int32)])
            qy = plsc.load_gather(q_all, [jnp.full((16,), qb + 1, jnp.int32)])
            qz = plsc.load_gather(q_all, [jnp.full((16,), qb + 2, jnp.int32)])
            nx0 = plsc.load_gather(xs_v, [iv0]) - qx
            ny0 = plsc.load_gather(ys_v, [iv0]) - qy
            nz0 = plsc.load_gather(zs_v, [iv0]) - qz
            nx1 = plsc.load_gather(xs_v, [iv1]) - qx
            ny1 = plsc.load_gather(ys_v, [iv1]) - qy
            nz1 = plsc.load_gather(zs_v, [iv1]) - qz
            bd0 = jnp.full((16,), 1e30, jnp.float32)
            bd1 = jnp.full((16,), 1e30, jnp.float32)
            bk0 = jnp.zeros((16,), jnp.int32)
            bk1 = jnp.zeros((16,), jnp.int32)
            for k in range(K):
                dx0 = nx0 - kpx[k]
                dy0 = ny0 - kpy[k]
                dz0 = nz0 - kpz[k]
                d0 = dx0 * dx0 + dy0 * dy0 + dz0 * dz0
                m0 = d0 < bd0
                bd0 = jnp.where(m0, d0, bd0)
                bk0 = jnp.where(m0, k, bk0)
                dx1 = nx1 - kpx[k]
                dy1 = ny1 - kpy[k]
                dz1 = nz1 - kpz[k]
                d1 = dx1 * dx1 + dy1 * dy1 + dz1 * dz1
                m1 = d1 < bd1
                bd1 = jnp.where(m1, d1, bd1)
                bk1 = jnp.where(m1, k, bk1)
            w0 = jnp.maximum(1.0 - _sqrt_approx(bd0) * (1.0 / SIGMA), 0.0)
            w1 = jnp.maximum(1.0 - _sqrt_approx(bd1) * (1.0 / SIGMA), 0.0)
            plsc.store_scatter(bk_scr, [lane + r * H], bk0)
            plsc.store_scatter(bk_scr, [lane + (r * H + 16)], bk1)
            plsc.store_scatter(w_scr, [lane + r * H], w0)
            plsc.store_scatter(w_scr, [lane + (r * H + 16)], w1)

        slanes_i = [lane * G + i for i in range(G)]

        def aggr_row(r, obase):
            bk0 = bk_scr[pl.ds(r * H, 16)]
            bk1 = bk_scr[pl.ds(r * H + 16, 16)]
            w0 = w_scr[pl.ds(r * H, 16)]
            w1 = w_scr[pl.ds(r * H + 16, 16)]
            acc = [jnp.zeros((16,), jnp.float32) for _ in range(G)]
            for h in range(H):
                if h < 16:
                    kh, wh = bk0[h], w0[h]
                else:
                    kh, wh = bk1[h - 16], w1[h - 16]
                mvec = mod_r[pl.ds(r * K * CPG + kh * CPG, CPG)]
                wm = wh * mvec
                wbase = kh * C
                rrow = jnp.full((16,), r * H + h, jnp.int32)
                for i in range(G):
                    fvec = plsc.load_gather(rows_v, [rrow, slanes_i[i]])
                    acc[i] = acc[i] \
                        + fvec * (wp_v[pl.ds(wbase + i * 16, 16)] * wm)
            for i in range(G):
                plsc.store_scatter(out_v, [slanes_i[i] + (obase + r * C)],
                                   acc[i])

        def chunk_body(ci, _):
            handle = pltpu.async_copy(
                feats_hbm.at[idx_all.at[pl.ds(ci * CHUNK * H, CHUNK * H)]],
                rows_v, gsem)
            mhandle = pltpu.async_copy(
                mod_hbm.at[pl.ds((base + ci * CHUNK) * K * CPG, modw)],
                mod_r, msem)

            def geom_body(r, _):
                geom_row(ci, r)
                return _
            lax.fori_loop(0, CHUNK, geom_body, None)

            @pl.when(ci >= 2)
            def _drain_o():
                pltpu.make_async_copy(
                    out_v.at[pl.ds(0, CHUNK * C)],
                    out_hbm.at[pl.ds(0, CHUNK * C)], osem).wait()

            mhandle.wait()
            handle.wait()

            obase = (ci % 2) * CHUNK * C

            def aggr_body(r, _):
                aggr_row(r, obase)
                return _
            lax.fori_loop(0, CHUNK, aggr_body, None)

            pltpu.async_copy(
                out_v.at[pl.ds(obase, CHUNK * C)],
                out_hbm.at[pl.ds((base + ci * CHUNK) * C, CHUNK * C)], osem)
            return _

        lax.fori_loop(0, n_chunks, chunk_body, None)

        pltpu.make_async_copy(out_v.at[pl.ds(0, CHUNK * C)],
                              out_hbm.at[pl.ds(0, CHUNK * C)], osem).wait()
        pltpu.make_async_copy(out_v.at[pl.ds(0, CHUNK * C)],
                              out_hbm.at[pl.ds(0, CHUNK * C)], osem).wait()

    return sc_kernel


def kernel(q_pts, s_pts, s_feats, neighb_inds, kernel_points, weights,
           W1, b1, W2):
    m, n = q_pts.shape[0], s_pts.shape[0]
    m_pad = ((m + NW * CHUNK - 1) // (NW * CHUNK)) * (NW * CHUNK)

    pad = m_pad - m
    s_feats_pad = jnp.pad(s_feats, ((0, pad), (0, 0)))
    mod = _run_mlp(s_feats_pad, W1, b1, W2)

    wp = weights.reshape(K, CPG, G).transpose(0, 2, 1).reshape(K * C)

    xs, ys, zs = s_pts[:, 0], s_pts[:, 1], s_pts[:, 2]
    kp = jnp.zeros((48,), jnp.float32)
    kp = kp.at[0:K].set(kernel_points[:, 0])
    kp = kp.at[16:16 + K].set(kernel_points[:, 1])
    kp = kp.at[32:32 + K].set(kernel_points[:, 2])

    inds_f = jnp.pad(neighb_inds.astype(jnp.int32), ((0, pad), (0, 0))
                     ).reshape(m_pad * H)
    q_f = jnp.pad(q_pts, ((0, pad), (0, 1))).reshape(m_pad * 4)
    mod_f = mod.reshape(m_pad * K * CPG)

    sc = _make_sc_kernel(n, m_pad)
    out_p = sc(s_feats, inds_f, q_f, mod_f, xs, ys, zs, kp, wp)

    return out_p.reshape(m_pad, C)[:m]


# speedup vs baseline: 1.0568x; 1.0568x over previous
import functools

import jax
import jax.numpy as jnp
from jax import lax
from jax.experimental import pallas as pl
from jax.experimental.pallas import tpu as pltpu
from jax.experimental.pallas import tpu_sc as plsc

C = 128
K = 15
H = 32
G = 8
CPG = 16
SIGMA = 0.6
NC = 2
NS = 16
NW = NC * NS
CHUNK = 4
NSLOT = 2


def _mlp_body(f_ref, w1_ref, b1_ref, w2_ref, o_ref):
    x = jnp.dot(f_ref[...], w1_ref[...], preferred_element_type=jnp.float32)
    x = x + b1_ref[...]
    x = jnp.where(x > 0, x, 0.1 * x)
    y = jnp.dot(x, w2_ref[...], preferred_element_type=jnp.float32)
    o_ref[...] = 1.0 / (1.0 + jnp.exp(-y))


def _run_mlp(s_feats, W1, b1, W2):
    m = s_feats.shape[0]
    bm = next(b for b in (1024, 512, 256, 128, 8) if m % b == 0)
    grid = (m // bm,)
    return pl.pallas_call(
        _mlp_body,
        grid=grid,
        in_specs=[
            pl.BlockSpec((bm, C), lambda i: (i, 0)),
            pl.BlockSpec((C, C), lambda i: (0, 0)),
            pl.BlockSpec((1, C), lambda i: (0, 0)),
            pl.BlockSpec((C, K * CPG), lambda i: (0, 0)),
        ],
        out_specs=pl.BlockSpec((bm, K * CPG), lambda i: (i, 0)),
        out_shape=jax.ShapeDtypeStruct((m, K * CPG), jnp.float32),
    )(s_feats, W1, b1.reshape(1, C), W2)


def _sqrt_approx(x):
    ix = plsc.bitcast(x, jnp.int32)
    y = plsc.bitcast(lax.shift_right_logical(ix, 1) + 0x1FBD1DF6, jnp.float32)
    y = 0.5 * (y + x / y)
    y = 0.5 * (y + x / y)
    y = 0.5 * (y + x / y)
    return y


def _make_sc_kernel(n_pts, m_pad):
    rows_per_tile = m_pad // NW
    n_chunks = rows_per_tile // CHUNK
    n_outer = n_chunks // NSLOT
    modw = CHUNK * K * CPG
    mesh = plsc.VectorSubcoreMesh(core_axis_name="c", subcore_axis_name="s")

    @functools.partial(
        pl.kernel,
        mesh=mesh,
        out_type=jax.ShapeDtypeStruct((m_pad * C,), jnp.float32),
        compiler_params=pltpu.CompilerParams(needs_layout_passes=False),
        scratch_types=[
            pltpu.VMEM((n_pts,), jnp.float32),
            pltpu.VMEM((n_pts,), jnp.float32),
            pltpu.VMEM((n_pts,), jnp.float32),
            pltpu.VMEM((48,), jnp.float32),
            pltpu.VMEM((K * C,), jnp.float32),
            pltpu.VMEM((rows_per_tile * 4,), jnp.float32),
            pltpu.VMEM((rows_per_tile * H,), jnp.int32),
            pltpu.VMEM((CHUNK * H, C), jnp.float32),
            pltpu.VMEM((modw,), jnp.float32),
            pltpu.VMEM((2 * CHUNK * C,), jnp.float32),
            pltpu.VMEM((CHUNK * H,), jnp.int32),
            pltpu.VMEM((CHUNK * H,), jnp.float32),
            pltpu.SemaphoreType.DMA,
            pltpu.SemaphoreType.DMA,
            pltpu.SemaphoreType.DMA,
        ],
    )
    def sc_kernel(feats_hbm, inds_hbm, q_hbm, mod_hbm, xs_hbm, ys_hbm,
                  zs_hbm, kp_hbm, wp_hbm, out_hbm,
                  xs_v, ys_v, zs_v, kp_v, wp_v, q_all, idx_all,
                  rows_v, mod_r, out_v, bk_scr, w_scr, gsem, msem, osem):
        wid = lax.axis_index("s") * NC + lax.axis_index("c")
        base = wid * rows_per_tile

        pltpu.sync_copy(xs_hbm, xs_v)
        pltpu.sync_copy(ys_hbm, ys_v)
        pltpu.sync_copy(zs_hbm, zs_v)
        pltpu.sync_copy(kp_hbm, kp_v)
        pltpu.sync_copy(wp_hbm, wp_v)
        pltpu.sync_copy(q_hbm.at[pl.ds(base * 4, rows_per_tile * 4)], q_all)
        pltpu.sync_copy(inds_hbm.at[pl.ds(base * H, rows_per_tile * H)],
                        idx_all)

        kpxv = kp_v[pl.ds(0, 16)]
        kpyv = kp_v[pl.ds(16, 16)]
        kpzv = kp_v[pl.ds(32, 16)]
        kpx = [kpxv[k] for k in range(K)]
        kpy = [kpyv[k] for k in range(K)]
        kpz = [kpzv[k] for k in range(K)]

        lane = lax.iota(jnp.int32, 16)

        def geom_row(ci, r):
            ib = ci * CHUNK * H + r * H
            iv0 = idx_all[pl.ds(ib, 16)]
            iv1 = idx_all[pl.ds(ib + 16, 16)]
            qb = ci * CHUNK * 4 + r * 4
            qx = plsc.load_gather(q_all, [jnp.full((16,), qb, jnp.int32)])
            qy = plsc.load_gather(q_all, [jnp.full((16,), qb + 1, jnp.int32)])
            qz = plsc.load_gather(q_all, [jnp.full((16,), qb + 2, jnp.int32)])
            nx0 = plsc.load_gather(xs_v, [iv0]) - qx
            ny0 = plsc.load_gather(ys_v, [iv0]) - qy
            nz0 = plsc.load_gather(zs_v, [iv0]) - qz
            nx1 = plsc.load_gather(xs_v, [iv1]) - qx
            ny1 = plsc.load_gather(ys_v, [iv1]) - qy
            nz1 = plsc.load_gather(zs_v, [iv1]) - qz
            bd0 = jnp.full((16,), 1e30, jnp.float32)
            bd1 = jnp.full((16,), 1e30, jnp.float32)
            bk0 = jnp.zeros((16,), jnp.int32)
            bk1 = jnp.zeros((16,), jnp.int32)
            for k in range(K):
                dx0 = nx0 - kpx[k]
                dy0 = ny0 - kpy[k]
                dz0 = nz0 - kpz[k]
                d0 = dx0 * dx0 + dy0 * dy0 + dz0 * dz0
                m0 = d0 < bd0
                bd0 = jnp.where(m0, d0, bd0)
                bk0 = jnp.where(m0, k, bk0)
                dx1 = nx1 - kpx[k]
                dy1 = ny1 - kpy[k]
                dz1 = nz1 - kpz[k]
                d1 = dx1 * dx1 + dy1 * dy1 + dz1 * dz1
                m1 = d1 < bd1
                bd1 = jnp.where(m1, d1, bd1)
                bk1 = jnp.where(m1, k, bk1)
            w0 = jnp.maximum(1.0 - _sqrt_approx(bd0) * (1.0 / SIGMA), 0.0)
            w1 = jnp.maximum(1.0 - _sqrt_approx(bd1) * (1.0 / SIGMA), 0.0)
            plsc.store_scatter(bk_scr, [lane + r * H], bk0)
            plsc.store_scatter(bk_scr, [lane + (r * H + 16)], bk1)
            plsc.store_scatter(w_scr, [lane + r * H], w0)
            plsc.store_scatter(w_scr, [lane + (r * H + 16)], w1)

        lanes_i = [lane + i * 16 for i in range(G)]
        slanes_i = [lane * G + i for i in range(G)]

        def aggr_row(r, obase):
            bk0 = bk_scr[pl.ds(r * H, 16)]
            bk1 = bk_scr[pl.ds(r * H + 16, 16)]
            w0 = w_scr[pl.ds(r * H, 16)]
            w1 = w_scr[pl.ds(r * H + 16, 16)]
            acc = [jnp.zeros((16,), jnp.float32) for _ in range(G)]
            for h in range(H):
                if h < 16:
                    kh, wh = bk0[h], w0[h]
                else:
                    kh, wh = bk1[h - 16], w1[h - 16]
                mvec = mod_r[pl.ds(r * K * CPG + kh * CPG, CPG)]
                wm = wh * mvec
                wbase = kh * C
                rrow = jnp.full((16,), r * H + h, jnp.int32)
                for i in range(G):
                    fvec = plsc.load_gather(rows_v, [rrow, lanes_i[i]])
                    acc[i] = acc[i] \
                        + fvec * (wp_v[pl.ds(wbase + i * 16, 16)] * wm)
            for i in range(G):
                plsc.store_scatter(out_v, [slanes_i[i] + (obase + r * C)],
                                   acc[i])

        def chunk_body(ci, _):
            handle = pltpu.async_copy(
                feats_hbm.at[idx_all.at[pl.ds(ci * CHUNK * H, CHUNK * H)]],
                rows_v, gsem)
            mhandle = pltpu.async_copy(
                mod_hbm.at[pl.ds((base + ci * CHUNK) * K * CPG, modw)],
                mod_r, msem)

            def geom_body(r, _):
                geom_row(ci, r)
                return _
            lax.fori_loop(0, CHUNK, geom_body, None)

            @pl.when(ci >= 2)
            def _drain_o():
                pltpu.make_async_copy(
                    out_v.at[pl.ds(0, CHUNK * C)],
                    out_hbm.at[pl.ds(0, CHUNK * C)], osem).wait()

            mhandle.wait()
            handle.wait()

            obase = (ci % 2) * CHUNK * C

            def aggr_body(r, _):
                aggr_row(r, obase)
                return _
            lax.fori_loop(0, CHUNK, aggr_body, None)

            pltpu.async_copy(
                out_v.at[pl.ds(obase, CHUNK * C)],
                out_hbm.at[pl.ds((base + ci * CHUNK) * C, CHUNK * C)], osem)
            return _

        lax.fori_loop(0, n_chunks, chunk_body, None)

        pltpu.make_async_copy(out_v.at[pl.ds(0, CHUNK * C)],
                              out_hbm.at[pl.ds(0, CHUNK * C)], osem).wait()
        pltpu.make_async_copy(out_v.at[pl.ds(0, CHUNK * C)],
                              out_hbm.at[pl.ds(0, CHUNK * C)], osem).wait()

    return sc_kernel


def kernel(q_pts, s_pts, s_feats, neighb_inds, kernel_points, weights,
           W1, b1, W2):
    m, n = q_pts.shape[0], s_pts.shape[0]
    m_pad = ((m + NW * CHUNK - 1) // (NW * CHUNK)) * (NW * CHUNK)

    pad = m_pad - m
    s_feats_pad = jnp.pad(s_feats, ((0, pad), (0, 0)))
    mod = _run_mlp(s_feats_pad, W1, b1, W2)

    feats_p = s_feats.reshape(n, CPG, G).transpose(0, 2, 1).reshape(n, C)
    wp = weights.reshape(K, CPG, G).transpose(0, 2, 1).reshape(K * C)

    xs, ys, zs = s_pts[:, 0], s_pts[:, 1], s_pts[:, 2]
    kp = jnp.zeros((48,), jnp.float32)
    kp = kp.at[0:K].set(kernel_points[:, 0])
    kp = kp.at[16:16 + K].set(kernel_points[:, 1])
    kp = kp.at[32:32 + K].set(kernel_points[:, 2])

    inds_f = jnp.pad(neighb_inds.astype(jnp.int32), ((0, pad), (0, 0))
                     ).reshape(m_pad * H)
    q_f = jnp.pad(q_pts, ((0, pad), (0, 1))).reshape(m_pad * 4)
    mod_f = mod.reshape(m_pad * K * CPG)

    sc = _make_sc_kernel(n, m_pad)
    out_p = sc(feats_p, inds_f, q_f, mod_f, xs, ys, zs, kp, wp)

    return out_p.reshape(m_pad, C)[:m]


# speedup vs baseline: 1.0588x; 1.0020x over previous
import functools

import jax
import jax.numpy as jnp
from jax import lax
from jax.experimental import pallas as pl
from jax.experimental.pallas import tpu as pltpu
from jax.experimental.pallas import tpu_sc as plsc

C = 128
K = 15
H = 32
G = 8
CPG = 16
SIGMA = 0.6
NC = 2
NS = 16
NW = NC * NS
CHUNK = 4
NSLOT = 2


def _mlp_body(f_ref, w1_ref, b1_ref, w2_ref, o_ref):
    x = jnp.dot(f_ref[...], w1_ref[...], preferred_element_type=jnp.float32)
    x = x + b1_ref[...]
    x = jnp.where(x > 0, x, 0.1 * x)
    y = jnp.dot(x, w2_ref[...], preferred_element_type=jnp.float32)
    o_ref[...] = 1.0 / (1.0 + jnp.exp(-y))


def _run_mlp(s_feats, W1, b1, W2):
    m = s_feats.shape[0]
    bm = next(b for b in (1024, 512, 256, 128, 8) if m % b == 0)
    grid = (m // bm,)
    return pl.pallas_call(
        _mlp_body,
        grid=grid,
        in_specs=[
            pl.BlockSpec((bm, C), lambda i: (i, 0)),
            pl.BlockSpec((C, C), lambda i: (0, 0)),
            pl.BlockSpec((1, C), lambda i: (0, 0)),
            pl.BlockSpec((C, K * CPG), lambda i: (0, 0)),
        ],
        out_specs=pl.BlockSpec((bm, K * CPG), lambda i: (i, 0)),
        out_shape=jax.ShapeDtypeStruct((m, K * CPG), jnp.float32),
    )(s_feats, W1, b1.reshape(1, C), W2)


def _sqrt_approx(x):
    ix = plsc.bitcast(x, jnp.int32)
    y = plsc.bitcast(lax.shift_right_logical(ix, 1) + 0x1FBD1DF6, jnp.float32)
    y = 0.5 * (y + x / y)
    y = 0.5 * (y + x / y)
    y = 0.5 * (y + x / y)
    return y


def _make_sc_kernel(n_pts, m_pad):
    rows_per_tile = m_pad // NW
    n_chunks = rows_per_tile // CHUNK
    n_outer = n_chunks // NSLOT
    modw = CHUNK * K * CPG
    mesh = plsc.VectorSubcoreMesh(core_axis_name="c", subcore_axis_name="s")

    @functools.partial(
        pl.kernel,
        mesh=mesh,
        out_type=jax.ShapeDtypeStruct((m_pad * C,), jnp.float32),
        compiler_params=pltpu.CompilerParams(needs_layout_passes=False),
        scratch_types=[
            pltpu.VMEM((n_pts,), jnp.float32),
            pltpu.VMEM((n_pts,), jnp.float32),
            pltpu.VMEM((n_pts,), jnp.float32),
            pltpu.VMEM_SHARED((n_pts,), jnp.float32),
            pltpu.VMEM_SHARED((n_pts,), jnp.float32),
            pltpu.VMEM_SHARED((n_pts,), jnp.float32),
            pltpu.VMEM((48,), jnp.float32),
            pltpu.VMEM((K * C,), jnp.float32),
            pltpu.VMEM((rows_per_tile * 4,), jnp.float32),
            pltpu.VMEM((rows_per_tile * H,), jnp.int32),
            pltpu.VMEM((CHUNK * H, C), jnp.float32),
            pltpu.VMEM((modw,), jnp.float32),
            pltpu.VMEM((2 * CHUNK * C,), jnp.float32),
            pltpu.VMEM((CHUNK * H,), jnp.int32),
            pltpu.VMEM((CHUNK * H,), jnp.float32),
            pltpu.SemaphoreType.DMA,
            pltpu.SemaphoreType.DMA,
            pltpu.SemaphoreType.DMA,
        ],
    )
    def sc_kernel(feats_hbm, inds_hbm, q_hbm, mod_hbm, xs_hbm, ys_hbm,
                  zs_hbm, kp_hbm, wp_hbm, out_hbm,
                  xs_v, ys_v, zs_v, sh_x, sh_y, sh_z, kp_v, wp_v, q_all,
                  idx_all, rows_v, mod_r, out_v, bk_scr, w_scr,
                  gsem, msem, osem):
        sid = lax.axis_index("s")
        wid = sid * NC + lax.axis_index("c")
        base = wid * rows_per_tile

        @pl.when(sid == 0)
        def _stage():
            pltpu.sync_copy(xs_hbm, sh_x)
            pltpu.sync_copy(ys_hbm, sh_y)
            pltpu.sync_copy(zs_hbm, sh_z)
        plsc.subcore_barrier()
        pltpu.sync_copy(sh_x, xs_v)
        pltpu.sync_copy(sh_y, ys_v)
        pltpu.sync_copy(sh_z, zs_v)
        pltpu.sync_copy(kp_hbm, kp_v)
        pltpu.sync_copy(wp_hbm, wp_v)
        pltpu.sync_copy(q_hbm.at[pl.ds(base * 4, rows_per_tile * 4)], q_all)
        pltpu.sync_copy(inds_hbm.at[pl.ds(base * H, rows_per_tile * H)],
                        idx_all)

        kpxv = kp_v[pl.ds(0, 16)]
        kpyv = kp_v[pl.ds(16, 16)]
        kpzv = kp_v[pl.ds(32, 16)]
        kpx = [kpxv[k] for k in range(K)]
        kpy = [kpyv[k] for k in range(K)]
        kpz = [kpzv[k] for k in range(K)]

        lane = lax.iota(jnp.int32, 16)

        def geom_row(ci, r):
            ib = ci * CHUNK * H + r * H
            iv0 = idx_all[pl.ds(ib, 16)]
            iv1 = idx_all[pl.ds(ib + 16, 16)]
            qb = ci * CHUNK * 4 + r * 4
            qx = plsc.load_gather(q_all, [jnp.full((16,), qb, jnp.int32)])
            qy = plsc.load_gather(q_all, [jnp.full((16,), qb + 1, jnp.int32)])
            qz = plsc.load_gather(q_all, [jnp.full((16,), qb + 2, jnp.int32)])
            nx0 = plsc.load_gather(xs_v, [iv0]) - qx
            ny0 = plsc.load_gather(ys_v, [iv0]) - qy
            nz0 = plsc.load_gather(zs_v, [iv0]) - qz
            nx1 = plsc.load_gather(xs_v, [iv1]) - qx
            ny1 = plsc.load_gather(ys_v, [iv1]) - qy
            nz1 = plsc.load_gather(zs_v, [iv1]) - qz
            bd0 = jnp.full((16,), 1e30, jnp.float32)
            bd1 = jnp.full((16,), 1e30, jnp.float32)
            bk0 = jnp.zeros((16,), jnp.int32)
            bk1 = jnp.zeros((16,), jnp.int32)
            for k in range(K):
                dx0 = nx0 - kpx[k]
                dy0 = ny0 - kpy[k]
                dz0 = nz0 - kpz[k]
                d0 = dx0 * dx0 + dy0 * dy0 + dz0 * dz0
                m0 = d0 < bd0
                bd0 = jnp.where(m0, d0, bd0)
                bk0 = jnp.where(m0, k, bk0)
                dx1 = nx1 - kpx[k]
                dy1 = ny1 - kpy[k]
                dz1 = nz1 - kpz[k]
                d1 = dx1 * dx1 + dy1 * dy1 + dz1 * dz1
                m1 = d1 < bd1
                bd1 = jnp.where(m1, d1, bd1)
                bk1 = jnp.where(m1, k, bk1)
            w0 = jnp.maximum(1.0 - _sqrt_approx(bd0) * (1.0 / SIGMA), 0.0)
            w1 = jnp.maximum(1.0 - _sqrt_approx(bd1) * (1.0 / SIGMA), 0.0)
            plsc.store_scatter(bk_scr, [lane + r * H], bk0)
            plsc.store_scatter(bk_scr, [lane + (r * H + 16)], bk1)
            plsc.store_scatter(w_scr, [lane + r * H], w0)
            plsc.store_scatter(w_scr, [lane + (r * H + 16)], w1)

        lanes_i = [lane + i * 16 for i in range(G)]
        slanes_i = [lane * G + i for i in range(G)]

        def aggr_row(r, obase):
            bk0 = bk_scr[pl.ds(r * H, 16)]
            bk1 = bk_scr[pl.ds(r * H + 16, 16)]
            w0 = w_scr[pl.ds(r * H, 16)]
            w1 = w_scr[pl.ds(r * H + 16, 16)]
            acc = [jnp.zeros((16,), jnp.float32) for _ in range(G)]
            for h in range(H):
                if h < 16:
                    kh, wh = bk0[h], w0[h]
                else:
                    kh, wh = bk1[h - 16], w1[h - 16]
                mvec = mod_r[pl.ds(r * K * CPG + kh * CPG, CPG)]
                wm = wh * mvec
                wbase = kh * C
                rrow = jnp.full((16,), r * H + h, jnp.int32)
                for i in range(G):
                    fvec = plsc.load_gather(rows_v, [rrow, lanes_i[i]])
                    acc[i] = acc[i] \
                        + fvec * (wp_v[pl.ds(wbase + i * 16, 16)] * wm)
            for i in range(G):
                plsc.store_scatter(out_v, [slanes_i[i] + (obase + r * C)],
                                   acc[i])

        def chunk_body(ci, _):
            handle = pltpu.async_copy(
                feats_hbm.at[idx_all.at[pl.ds(ci * CHUNK * H, CHUNK * H)]],
                rows_v, gsem)
            mhandle = pltpu.async_copy(
                mod_hbm.at[pl.ds((base + ci * CHUNK) * K * CPG, modw)],
                mod_r, msem)

            def geom_body(r, _):
                geom_row(ci, r)
                return _
            lax.fori_loop(0, CHUNK, geom_body, None)

            @pl.when(ci >= 2)
            def _drain_o():
                pltpu.make_async_copy(
                    out_v.at[pl.ds(0, CHUNK * C)],
                    out_hbm.at[pl.ds(0, CHUNK * C)], osem).wait()

            mhandle.wait()
            handle.wait()

            obase = (ci % 2) * CHUNK * C

            def aggr_body(r, _):
                aggr_row(r, obase)
                return _
            lax.fori_loop(0, CHUNK, aggr_body, None)

            pltpu.async_copy(
                out_v.at[pl.ds(obase, CHUNK * C)],
                out_hbm.at[pl.ds((base + ci * CHUNK) * C, CHUNK * C)], osem)
            return _

        lax.fori_loop(0, n_chunks, chunk_body, None)

        pltpu.make_async_copy(out_v.at[pl.ds(0, CHUNK * C)],
                              out_hbm.at[pl.ds(0, CHUNK * C)], osem).wait()
        pltpu.make_async_copy(out_v.at[pl.ds(0, CHUNK * C)],
                              out_hbm.at[pl.ds(0, CHUNK * C)], osem).wait()

    return sc_kernel


def kernel(q_pts, s_pts, s_feats, neighb_inds, kernel_points, weights,
           W1, b1, W2):
    m, n = q_pts.shape[0], s_pts.shape[0]
    m_pad = ((m + NW * CHUNK - 1) // (NW * CHUNK)) * (NW * CHUNK)

    pad = m_pad - m
    s_feats_pad = jnp.pad(s_feats, ((0, pad), (0, 0)))
    mod = _run_mlp(s_feats_pad, W1, b1, W2)

    feats_p = s_feats.reshape(n, CPG, G).transpose(0, 2, 1).reshape(n, C)
    wp = weights.reshape(K, CPG, G).transpose(0, 2, 1).reshape(K * C)

    xs, ys, zs = s_pts[:, 0], s_pts[:, 1], s_pts[:, 2]
    kp = jnp.zeros((48,), jnp.float32)
    kp = kp.at[0:K].set(kernel_points[:, 0])
    kp = kp.at[16:16 + K].set(kernel_points[:, 1])
    kp = kp.at[32:32 + K].set(kernel_points[:, 2])

    inds_f = jnp.pad(neighb_inds.astype(jnp.int32), ((0, pad), (0, 0))
                     ).reshape(m_pad * H)
    q_f = jnp.pad(q_pts, ((0, pad), (0, 1))).reshape(m_pad * 4)
    mod_f = mod.reshape(m_pad * K * CPG)

    sc = _make_sc_kernel(n, m_pad)
    out_p = sc(feats_p, inds_f, q_f, mod_f, xs, ys, zs, kp, wp)

    return out_p.reshape(m_pad, C)[:m]


# speedup vs baseline: 1.2791x; 1.2080x over previous
import functools

import jax
import jax.numpy as jnp
from jax import lax
from jax.experimental import pallas as pl
from jax.experimental.pallas import tpu as pltpu
from jax.experimental.pallas import tpu_sc as plsc

C = 128
K = 15
H = 32
G = 8
CPG = 16
SIGMA = 0.6
NC = 2
NS = 16
NW = NC * NS
CHUNK = 4
NSLOT = 2


def _mlp_body(f_ref, w1_ref, b1_ref, w2_ref, o_ref):
    x = jnp.dot(f_ref[...], w1_ref[...], preferred_element_type=jnp.float32)
    x = x + b1_ref[...]
    x = jnp.where(x > 0, x, 0.1 * x)
    y = jnp.dot(x, w2_ref[...], preferred_element_type=jnp.float32)
    o_ref[...] = 1.0 / (1.0 + jnp.exp(-y))


def _run_mlp(s_feats, W1, b1, W2):
    m = s_feats.shape[0]
    bm = next(b for b in (1024, 512, 256, 128, 8) if m % b == 0)
    grid = (m // bm,)
    return pl.pallas_call(
        _mlp_body,
        grid=grid,
        in_specs=[
            pl.BlockSpec((bm, C), lambda i: (i, 0)),
            pl.BlockSpec((C, C), lambda i: (0, 0)),
            pl.BlockSpec((1, C), lambda i: (0, 0)),
            pl.BlockSpec((C, K * CPG), lambda i: (0, 0)),
        ],
        out_specs=pl.BlockSpec((bm, K * CPG), lambda i: (i, 0)),
        out_shape=jax.ShapeDtypeStruct((m, K * CPG), jnp.float32),
    )(s_feats, W1, b1.reshape(1, C), W2)


def _sqrt_approx(x):
    ix = plsc.bitcast(x, jnp.int32)
    y = plsc.bitcast(lax.shift_right_logical(ix, 1) + 0x1FBD1DF6, jnp.float32)
    y = 0.5 * (y + x / y)
    y = 0.5 * (y + x / y)
    y = 0.5 * (y + x / y)
    return y


def _make_sc_kernel(n_pts, m_pad):
    rows_per_tile = m_pad // NW
    n_chunks = rows_per_tile // CHUNK
    n_outer = n_chunks // NSLOT
    modw = CHUNK * K * CPG
    mesh = plsc.VectorSubcoreMesh(core_axis_name="c", subcore_axis_name="s")

    @functools.partial(
        pl.kernel,
        mesh=mesh,
        out_type=jax.ShapeDtypeStruct((m_pad * C,), jnp.float32),
        compiler_params=pltpu.CompilerParams(needs_layout_passes=False),
        scratch_types=[
            pltpu.VMEM((n_pts,), jnp.float32),
            pltpu.VMEM((n_pts,), jnp.float32),
            pltpu.VMEM((n_pts,), jnp.float32),
            pltpu.VMEM_SHARED((n_pts,), jnp.float32),
            pltpu.VMEM_SHARED((n_pts,), jnp.float32),
            pltpu.VMEM_SHARED((n_pts,), jnp.float32),
            pltpu.VMEM((48,), jnp.float32),
            pltpu.VMEM((K * C,), jnp.float32),
            pltpu.VMEM((rows_per_tile * 4,), jnp.float32),
            pltpu.VMEM((rows_per_tile * H,), jnp.int32),
            pltpu.VMEM((2 * CHUNK * H, C), jnp.float32),
            pltpu.VMEM((modw,), jnp.float32),
            pltpu.VMEM((2 * CHUNK * C,), jnp.float32),
            pltpu.VMEM((CHUNK * H,), jnp.int32),
            pltpu.VMEM((CHUNK * H,), jnp.float32),
            pltpu.SemaphoreType.DMA,
            pltpu.SemaphoreType.DMA,
            pltpu.SemaphoreType.DMA,
        ],
    )
    def sc_kernel(feats_hbm, inds_hbm, q_hbm, mod_hbm, xs_hbm, ys_hbm,
                  zs_hbm, kp_hbm, wp_hbm, out_hbm,
                  xs_v, ys_v, zs_v, sh_x, sh_y, sh_z, kp_v, wp_v, q_all,
                  idx_all, rows_v, mod_r, out_v, bk_scr, w_scr,
                  gsem, msem, osem):
        sid = lax.axis_index("s")
        wid = sid * NC + lax.axis_index("c")
        base = wid * rows_per_tile

        @pl.when(sid == 0)
        def _stage():
            pltpu.sync_copy(xs_hbm, sh_x)
            pltpu.sync_copy(ys_hbm, sh_y)
            pltpu.sync_copy(zs_hbm, sh_z)
        plsc.subcore_barrier()
        pltpu.sync_copy(sh_x, xs_v)
        pltpu.sync_copy(sh_y, ys_v)
        pltpu.sync_copy(sh_z, zs_v)
        pltpu.sync_copy(kp_hbm, kp_v)
        pltpu.sync_copy(wp_hbm, wp_v)
        pltpu.sync_copy(q_hbm.at[pl.ds(base * 4, rows_per_tile * 4)], q_all)
        pltpu.sync_copy(inds_hbm.at[pl.ds(base * H, rows_per_tile * H)],
                        idx_all)

        kpxv = kp_v[pl.ds(0, 16)]
        kpyv = kp_v[pl.ds(16, 16)]
        kpzv = kp_v[pl.ds(32, 16)]
        kpx = [kpxv[k] for k in range(K)]
        kpy = [kpyv[k] for k in range(K)]
        kpz = [kpzv[k] for k in range(K)]

        lane = lax.iota(jnp.int32, 16)

        def geom_row(ci, r):
            ib = ci * CHUNK * H + r * H
            iv0 = idx_all[pl.ds(ib, 16)]
            iv1 = idx_all[pl.ds(ib + 16, 16)]
            qb = ci * CHUNK * 4 + r * 4
            qx = plsc.load_gather(q_all, [jnp.full((16,), qb, jnp.int32)])
            qy = plsc.load_gather(q_all, [jnp.full((16,), qb + 1, jnp.int32)])
            qz = plsc.load_gather(q_all, [jnp.full((16,), qb + 2, jnp.int32)])
            nx0 = plsc.load_gather(xs_v, [iv0]) - qx
            ny0 = plsc.load_gather(ys_v, [iv0]) - qy
            nz0 = plsc.load_gather(zs_v, [iv0]) - qz
            nx1 = plsc.load_gather(xs_v, [iv1]) - qx
            ny1 = plsc.load_gather(ys_v, [iv1]) - qy
            nz1 = plsc.load_gather(zs_v, [iv1]) - qz
            bd0 = jnp.full((16,), 1e30, jnp.float32)
            bd1 = jnp.full((16,), 1e30, jnp.float32)
            bk0 = jnp.zeros((16,), jnp.int32)
            bk1 = jnp.zeros((16,), jnp.int32)
            for k in range(K):
                dx0 = nx0 - kpx[k]
                dy0 = ny0 - kpy[k]
                dz0 = nz0 - kpz[k]
                d0 = dx0 * dx0 + dy0 * dy0 + dz0 * dz0
                m0 = d0 < bd0
                bd0 = jnp.where(m0, d0, bd0)
                bk0 = jnp.where(m0, k, bk0)
                dx1 = nx1 - kpx[k]
                dy1 = ny1 - kpy[k]
                dz1 = nz1 - kpz[k]
                d1 = dx1 * dx1 + dy1 * dy1 + dz1 * dz1
                m1 = d1 < bd1
                bd1 = jnp.where(m1, d1, bd1)
                bk1 = jnp.where(m1, k, bk1)
            w0 = jnp.maximum(1.0 - _sqrt_approx(bd0) * (1.0 / SIGMA), 0.0)
            w1 = jnp.maximum(1.0 - _sqrt_approx(bd1) * (1.0 / SIGMA), 0.0)
            plsc.store_scatter(bk_scr, [lane + r * H], bk0)
            plsc.store_scatter(bk_scr, [lane + (r * H + 16)], bk1)
            plsc.store_scatter(w_scr, [lane + r * H], w0)
            plsc.store_scatter(w_scr, [lane + (r * H + 16)], w1)

        lanes_i = [lane + i * 16 for i in range(G)]
        slanes_i = [lane * G + i for i in range(G)]

        def aggr_row(r, obase, rbase):
            bk0 = bk_scr[pl.ds(r * H, 16)]
            bk1 = bk_scr[pl.ds(r * H + 16, 16)]
            w0 = w_scr[pl.ds(r * H, 16)]
            w1 = w_scr[pl.ds(r * H + 16, 16)]
            acc = [jnp.zeros((16,), jnp.float32) for _ in range(G)]
            for h in range(H):
                if h < 16:
                    kh, wh = bk0[h], w0[h]
                else:
                    kh, wh = bk1[h - 16], w1[h - 16]
                mvec = mod_r[pl.ds(r * K * CPG + kh * CPG, CPG)]
                wm = wh * mvec
                wbase = kh * C
                rrow = jnp.full((16,), rbase + r * H + h, jnp.int32)
                for i in range(G):
                    fvec = plsc.load_gather(rows_v, [rrow, lanes_i[i]])
                    acc[i] = acc[i] \
                        + fvec * (wp_v[pl.ds(wbase + i * 16, 16)] * wm)
            for i in range(G):
                plsc.store_scatter(out_v, [slanes_i[i] + (obase + r * C)],
                                   acc[i])

        def gather_descr(ci):
            return pltpu.make_async_copy(
                feats_hbm.at[idx_all.at[pl.ds(ci * CHUNK * H, CHUNK * H)]],
                rows_v.at[pl.ds(pl.multiple_of((ci % 2) * CHUNK * H,
                                               CHUNK * H), CHUNK * H), :],
                gsem)

        gather_descr(0).start()

        def chunk_body(ci, _):
            @pl.when(ci + 1 < n_chunks)
            def _prefetch():
                gather_descr(ci + 1).start()
            mhandle = pltpu.async_copy(
                mod_hbm.at[pl.ds((base + ci * CHUNK) * K * CPG, modw)],
                mod_r, msem)

            def geom_body(r, _):
                geom_row(ci, r)
                return _
            lax.fori_loop(0, CHUNK, geom_body, None)

            @pl.when(ci >= 2)
            def _drain_o():
                pltpu.make_async_copy(
                    out_v.at[pl.ds(0, CHUNK * C)],
                    out_hbm.at[pl.ds(0, CHUNK * C)], osem).wait()

            mhandle.wait()
            gather_descr(ci).wait()

            obase = (ci % 2) * CHUNK * C
            rbase = (ci % 2) * CHUNK * H

            def aggr_body(r, _):
                aggr_row(r, obase, rbase)
                return _
            lax.fori_loop(0, CHUNK, aggr_body, None)

            pltpu.async_copy(
                out_v.at[pl.ds(obase, CHUNK * C)],
                out_hbm.at[pl.ds((base + ci * CHUNK) * C, CHUNK * C)], osem)
            return _

        lax.fori_loop(0, n_chunks, chunk_body, None)

        pltpu.make_async_copy(out_v.at[pl.ds(0, CHUNK * C)],
                              out_hbm.at[pl.ds(0, CHUNK * C)], osem).wait()
        pltpu.make_async_copy(out_v.at[pl.ds(0, CHUNK * C)],
                              out_hbm.at[pl.ds(0, CHUNK * C)], osem).wait()

    return sc_kernel


def kernel(q_pts, s_pts, s_feats, neighb_inds, kernel_points, weights,
           W1, b1, W2):
    m, n = q_pts.shape[0], s_pts.shape[0]
    m_pad = ((m + NW * CHUNK - 1) // (NW * CHUNK)) * (NW * CHUNK)

    pad = m_pad - m
    s_feats_pad = jnp.pad(s_feats, ((0, pad), (0, 0)))
    mod = _run_mlp(s_feats_pad, W1, b1, W2)

    feats_p = s_feats.reshape(n, CPG, G).transpose(0, 2, 1).reshape(n, C)
    wp = weights.reshape(K, CPG, G).transpose(0, 2, 1).reshape(K * C)

    xs, ys, zs = s_pts[:, 0], s_pts[:, 1], s_pts[:, 2]
    kp = jnp.zeros((48,), jnp.float32)
    kp = kp.at[0:K].set(kernel_points[:, 0])
    kp = kp.at[16:16 + K].set(kernel_points[:, 1])
    kp = kp.at[32:32 + K].set(kernel_points[:, 2])

    inds_f = jnp.pad(neighb_inds.astype(jnp.int32), ((0, pad), (0, 0))
                     ).reshape(m_pad * H)
    q_f = jnp.pad(q_pts, ((0, pad), (0, 1))).reshape(m_pad * 4)
    mod_f = mod.reshape(m_pad * K * CPG)

    sc = _make_sc_kernel(n, m_pad)
    out_p = sc(feats_p, inds_f, q_f, mod_f, xs, ys, zs, kp, wp)

    return out_p.reshape(m_pad, C)[:m]
